# baseline (speedup 1.0000x reference)
import dataclasses
import functools

import jax
import jax.numpy as jnp
from jax import lax
from jax.experimental import pallas as pl
from jax.experimental.pallas import tpu as pltpu
from jax.experimental.pallas import tpu_sc as plsc

NC = 2
NS = 16
L = 16
NW = NC * NS
B = 4096
H = 200
D = 64
W = 2 * D
BPW = B // NW
G0 = 128
G1 = H - G0
UNROLL = 8
REPACK_BLK = 20000


def _sc_compiler_params():
    cp = pltpu.CompilerParams()
    fields = pltpu.CompilerParams.__dataclass_fields__
    if "needs_layout_passes" in fields:
        cp = dataclasses.replace(cp, needs_layout_passes=False)
    if "use_tc_tiling_on_sc" in fields:
        cp = dataclasses.replace(cp, use_tc_tiling_on_sc=False)
    return cp


def _tc_widen_table(embed_w):
    v = embed_w.shape[0]

    def body(in_ref, o_ref):
        o_ref[:, :D] = in_ref[...]

    return pl.pallas_call(
        body,
        grid=(v // REPACK_BLK,),
        in_specs=[pl.BlockSpec((REPACK_BLK, D), lambda i: (i, 0))],
        out_specs=pl.BlockSpec((REPACK_BLK, W), lambda i: (i, 0)),
        out_shape=jax.ShapeDtypeStruct((v, W), jnp.float32),
    )(embed_w)


def _sc_embedding_bag(ctx, ctx_v, itm_flat, tab2):
    mesh = plsc.VectorSubcoreMesh(core_axis_name="c", subcore_axis_name="s")

    @functools.partial(
        pl.kernel,
        out_type=[jax.ShapeDtypeStruct((B, D), jnp.float32),
                  jax.ShapeDtypeStruct((B, W), jnp.float32)],
        mesh=mesh,
        compiler_params=_sc_compiler_params(),
        scratch_types=[
            pltpu.VMEM((BPW, H), jnp.int32),
            pltpu.VMEM((BPW, H), jnp.float32),
            pltpu.VMEM((H, W), jnp.float32),
            pltpu.VMEM((H, W), jnp.float32),
            pltpu.VMEM((BPW, D), jnp.float32),
            pltpu.VMEM((BPW,), jnp.int32),
            pltpu.VMEM((BPW, W), jnp.float32),
            pltpu.SemaphoreType.DMA,
            pltpu.SemaphoreType.DMA,
        ],
    )
    def k(ctx_hbm, ctxv_hbm, itm_hbm, tab_hbm, ctxsum_hbm, itmrows_hbm,
          idx_v, w_v, rows0, rows1, out_v, itmidx_v, itmrows_v, sem0, sem1):
        wid = lax.axis_index("s") * NC + lax.axis_index("c")
        base = wid * BPW

        pltpu.sync_copy(ctx_hbm.at[pl.ds(base, BPW)], idx_v)
        pltpu.sync_copy(ctxv_hbm.at[pl.ds(base, BPW)], w_v)

        def issue(e, buf, sem):
            pltpu.make_async_copy(
                tab_hbm.at[idx_v.at[e, pl.ds(0, G0)]],
                buf.at[pl.ds(0, G0)], sem).start()
            pltpu.make_async_copy(
                tab_hbm.at[idx_v.at[e, pl.ds(G0, G1)]],
                buf.at[pl.ds(G0, G1)], sem).start()

        def drain(e, buf, sem):
            pltpu.make_async_copy(
                tab_hbm.at[idx_v.at[e, pl.ds(0, G0)]],
                buf.at[pl.ds(0, G0)], sem).wait()
            pltpu.make_async_copy(
                tab_hbm.at[idx_v.at[e, pl.ds(G0, G1)]],
                buf.at[pl.ds(G0, G1)], sem).wait()

        def compute(e, buf):
            def body(l0, accs):
                for u in range(UNROLL):
                    l = l0 * UNROLL + u
                    wv = plsc.load_gather(
                        w_v, [jnp.full((L,), e, jnp.int32),
                              jnp.full((L,), l, jnp.int32)])
                    accs = tuple(acc + wv * buf[l, pl.ds(j * L, L)]
                                 for j, acc in enumerate(accs))
                return accs

            accs = lax.fori_loop(
                0, H // UNROLL, body,
                tuple(jnp.zeros((L,), jnp.float32) for _ in range(D // L)))
            for j in range(D // L):
                out_v[e, pl.ds(j * L, L)] = accs[j]

        issue(0, rows0, sem0)

        @pl.loop(0, BPW // 2)
        def _(p):
            e0 = p * 2
            issue(e0 + 1, rows1, sem1)
            drain(e0, rows0, sem0)
            compute(e0, rows0)
            issue(jnp.minimum(e0 + 2, BPW - 1), rows0, sem0)
            drain(e0 + 1, rows1, sem1)
            compute(e0 + 1, rows1)

        drain(BPW - 1, rows0, sem0)

        pltpu.sync_copy(out_v, ctxsum_hbm.at[pl.ds(base, BPW)])

        pltpu.sync_copy(itm_hbm.at[pl.ds(base, BPW)], itmidx_v)
        pltpu.sync_copy(tab_hbm.at[itmidx_v], itmrows_v)
        pltpu.sync_copy(itmrows_v, itmrows_hbm.at[pl.ds(base, BPW)])

    return k(ctx, ctx_v, itm_flat, tab2)


def _tc_tail(ctx_sum, itm_rows, z, fc1_w, fc1_b):
    def body(cs_ref, it_ref, z_ref, w_ref, b_ref, o_ref):
        p = (cs_ref[...] * it_ref[:, :D] * w_ref[:, :D]
             + z_ref[...] * w_ref[:, D:])
        o_ref[...] = jnp.sum(p, axis=1, keepdims=True) + b_ref[...]

    return pl.pallas_call(
        body,
        out_shape=jax.ShapeDtypeStruct((B, 1), jnp.float32),
    )(ctx_sum, itm_rows, z, fc1_w, fc1_b)


def kernel(ctx, itm, pos, ctx_v, z, embed_w, fc1_w, fc1_b):
    del pos
    tab2 = _tc_widen_table(embed_w)
    ctx_sum, itm_rows = _sc_embedding_bag(ctx, ctx_v, itm.reshape(B), tab2)
    return _tc_tail(ctx_sum, itm_rows, z, fc1_w, fc1_b.reshape(1, 1))


# speedup vs baseline: 1.1904x; 1.1904x over previous
import dataclasses
import functools

import jax
import jax.numpy as jnp
from jax import lax
from jax.experimental import pallas as pl
from jax.experimental.pallas import tpu as pltpu
from jax.experimental.pallas import tpu_sc as plsc

NC = 2
NS = 16
L = 16
NW = NC * NS
B = 4096
H = 200
D = 64
W = 2 * D
BPW = B // NW
G0 = 128
G1 = H - G0
UNROLL = 8
REPACK_BLK = 20000


def _sc_compiler_params():
    cp = pltpu.CompilerParams()
    fields = pltpu.CompilerParams.__dataclass_fields__
    if "needs_layout_passes" in fields:
        cp = dataclasses.replace(cp, needs_layout_passes=False)
    if "use_tc_tiling_on_sc" in fields:
        cp = dataclasses.replace(cp, use_tc_tiling_on_sc=False)
    return cp


def _tc_widen_table(embed_w):
    v = embed_w.shape[0]

    def body(in_ref, o_ref):
        o_ref[:, :D] = in_ref[...]

    return pl.pallas_call(
        body,
        grid=(v // REPACK_BLK,),
        in_specs=[pl.BlockSpec((REPACK_BLK, D), lambda i: (i, 0))],
        out_specs=pl.BlockSpec((REPACK_BLK, W), lambda i: (i, 0)),
        out_shape=jax.ShapeDtypeStruct((v, W), jnp.float32),
    )(embed_w)


def _sc_embedding_bag(ctx, ctx_v, itm_flat, tab2):
    mesh = plsc.VectorSubcoreMesh(core_axis_name="c", subcore_axis_name="s")

    @functools.partial(
        pl.kernel,
        out_type=[jax.ShapeDtypeStruct((B, D), jnp.float32),
                  jax.ShapeDtypeStruct((B, W), jnp.float32)],
        mesh=mesh,
        compiler_params=_sc_compiler_params(),
        scratch_types=[
            pltpu.VMEM((BPW, H), jnp.int32),
            pltpu.VMEM((BPW, H), jnp.float32),
            pltpu.VMEM((H, W), jnp.float32),
            pltpu.VMEM((H, W), jnp.float32),
            pltpu.VMEM((BPW, D), jnp.float32),
            pltpu.VMEM((BPW,), jnp.int32),
            pltpu.VMEM((BPW, W), jnp.float32),
            pltpu.SemaphoreType.DMA,
            pltpu.SemaphoreType.DMA,
        ],
    )
    def k(ctx_hbm, ctxv_hbm, itm_hbm, tab_hbm, ctxsum_hbm, itmrows_hbm,
          idx_v, w_v, rows0, rows1, out_v, itmidx_v, itmrows_v, sem0, sem1):
        wid = lax.axis_index("s") * NC + lax.axis_index("c")
        base = wid * BPW

        pltpu.sync_copy(ctx_hbm.at[pl.ds(base, BPW)], idx_v)
        pltpu.sync_copy(ctxv_hbm.at[pl.ds(base, BPW)], w_v)

        def issue(e, buf, sem):
            pltpu.make_async_copy(
                tab_hbm.at[idx_v.at[e, pl.ds(0, G0)]],
                buf.at[pl.ds(0, G0)], sem).start()
            pltpu.make_async_copy(
                tab_hbm.at[idx_v.at[e, pl.ds(G0, G1)]],
                buf.at[pl.ds(G0, G1)], sem).start()

        def drain(e, buf, sem):
            pltpu.make_async_copy(
                tab_hbm.at[idx_v.at[e, pl.ds(0, G0)]],
                buf.at[pl.ds(0, G0)], sem).wait()
            pltpu.make_async_copy(
                tab_hbm.at[idx_v.at[e, pl.ds(G0, G1)]],
                buf.at[pl.ds(G0, G1)], sem).wait()

        def compute(e, buf):
            def body(l0, accs):
                for u in range(UNROLL):
                    l = l0 * UNROLL + u
                    wv = plsc.load_gather(
                        w_v, [jnp.full((L,), e, jnp.int32),
                              jnp.full((L,), l, jnp.int32)])
                    accs = tuple(acc + wv * buf[l, pl.ds(j * L, L)]
                                 for j, acc in enumerate(accs))
                return accs

            accs = lax.fori_loop(
                0, H // UNROLL, body,
                tuple(jnp.zeros((L,), jnp.float32) for _ in range(D // L)))
            for j in range(D // L):
                out_v[e, pl.ds(j * L, L)] = accs[j]

        issue(0, rows0, sem0)

        @pl.loop(0, BPW // 2)
        def _(p):
            e0 = p * 2
            issue(e0 + 1, rows1, sem1)
            drain(e0, rows0, sem0)
            compute(e0, rows0)
            issue(jnp.minimum(e0 + 2, BPW - 1), rows0, sem0)
            drain(e0 + 1, rows1, sem1)
            compute(e0 + 1, rows1)

        drain(BPW - 1, rows0, sem0)

        pltpu.sync_copy(out_v, ctxsum_hbm.at[pl.ds(base, BPW)])

        pltpu.sync_copy(itm_hbm.at[pl.ds(base, BPW)], itmidx_v)
        pltpu.sync_copy(tab_hbm.at[itmidx_v], itmrows_v)
        pltpu.sync_copy(itmrows_v, itmrows_hbm.at[pl.ds(base, BPW)])

    return k(ctx, ctx_v, itm_flat, tab2)


def _tc_tail(ctx_sum, itm_rows, z, fc1_w, fc1_b):
    def body(cs_ref, it_ref, z_ref, w_ref, b_ref, o_ref):
        p = (cs_ref[...] * it_ref[:, :D] * w_ref[:, :D]
             + z_ref[...] * w_ref[:, D:])
        o_ref[...] = jnp.sum(p, axis=1, keepdims=True) + b_ref[...]

    return pl.pallas_call(
        body,
        out_shape=jax.ShapeDtypeStruct((B, 1), jnp.float32),
    )(ctx_sum, itm_rows, z, fc1_w, fc1_b)


def kernel(ctx, itm, pos, ctx_v, z, embed_w, fc1_w, fc1_b):
    del pos
    tab2 = jnp.pad(embed_w, ((0, 0), (0, D)))
    ctx_sum, itm_rows = _sc_embedding_bag(ctx, ctx_v, itm.reshape(B), tab2)
    return _tc_tail(ctx_sum, itm_rows, z, fc1_w, fc1_b.reshape(1, 1))


# speedup vs baseline: 1.5450x; 1.2979x over previous
import dataclasses
import functools

import jax
import jax.numpy as jnp
from jax import lax
from jax.experimental import pallas as pl
from jax.experimental.pallas import tpu as pltpu
from jax.experimental.pallas import tpu_sc as plsc

NC = 2
NS = 16
L = 16
NW = NC * NS
B = 4096
H = 200
D = 64
W = 2 * D
BPW = B // NW
G0 = 128
G1 = H - G0
UNROLL = 8
REPACK_BLK = 4096


def _sc_compiler_params():
    cp = pltpu.CompilerParams()
    fields = pltpu.CompilerParams.__dataclass_fields__
    if "needs_layout_passes" in fields:
        cp = dataclasses.replace(cp, needs_layout_passes=False)
    if "use_tc_tiling_on_sc" in fields:
        cp = dataclasses.replace(cp, use_tc_tiling_on_sc=False)
    return cp


def _tc_widen_table(embed_w):
    v = embed_w.shape[0]
    tab_t = embed_w.T

    def body(in_ref, o_ref):
        o_ref[:, :D] = in_ref[...].T

    return pl.pallas_call(
        body,
        grid=((v + REPACK_BLK - 1) // REPACK_BLK,),
        in_specs=[pl.BlockSpec((D, REPACK_BLK), lambda i: (0, i))],
        out_specs=pl.BlockSpec((REPACK_BLK, W), lambda i: (i, 0)),
        out_shape=jax.ShapeDtypeStruct((v, W), jnp.float32),
    )(tab_t)


def _sc_embedding_bag(ctx, ctx_v, itm_flat, tab2):
    mesh = plsc.VectorSubcoreMesh(core_axis_name="c", subcore_axis_name="s")

    @functools.partial(
        pl.kernel,
        out_type=[jax.ShapeDtypeStruct((B, D), jnp.float32),
                  jax.ShapeDtypeStruct((B, W), jnp.float32)],
        mesh=mesh,
        compiler_params=_sc_compiler_params(),
        scratch_types=[
            pltpu.VMEM((BPW, H), jnp.int32),
            pltpu.VMEM((BPW, H), jnp.float32),
            pltpu.VMEM((H, W), jnp.float32),
            pltpu.VMEM((H, W), jnp.float32),
            pltpu.VMEM((BPW, D), jnp.float32),
            pltpu.VMEM((BPW,), jnp.int32),
            pltpu.VMEM((BPW, W), jnp.float32),
            pltpu.SemaphoreType.DMA,
            pltpu.SemaphoreType.DMA,
        ],
    )
    def k(ctx_hbm, ctxv_hbm, itm_hbm, tab_hbm, ctxsum_hbm, itmrows_hbm,
          idx_v, w_v, rows0, rows1, out_v, itmidx_v, itmrows_v, sem0, sem1):
        wid = lax.axis_index("s") * NC + lax.axis_index("c")
        base = wid * BPW

        pltpu.sync_copy(ctx_hbm.at[pl.ds(base, BPW)], idx_v)
        pltpu.sync_copy(ctxv_hbm.at[pl.ds(base, BPW)], w_v)

        def issue(e, buf, sem):
            pltpu.make_async_copy(
                tab_hbm.at[idx_v.at[e, pl.ds(0, G0)]],
                buf.at[pl.ds(0, G0)], sem).start()
            pltpu.make_async_copy(
                tab_hbm.at[idx_v.at[e, pl.ds(G0, G1)]],
                buf.at[pl.ds(G0, G1)], sem).start()

        def drain(e, buf, sem):
            pltpu.make_async_copy(
                tab_hbm.at[idx_v.at[e, pl.ds(0, G0)]],
                buf.at[pl.ds(0, G0)], sem).wait()
            pltpu.make_async_copy(
                tab_hbm.at[idx_v.at[e, pl.ds(G0, G1)]],
                buf.at[pl.ds(G0, G1)], sem).wait()

        def compute(e, buf):
            def body(l0, accs):
                for u in range(UNROLL):
                    l = l0 * UNROLL + u
                    wv = plsc.load_gather(
                        w_v, [jnp.full((L,), e, jnp.int32),
                              jnp.full((L,), l, jnp.int32)])
                    accs = tuple(acc + wv * buf[l, pl.ds(j * L, L)]
                                 for j, acc in enumerate(accs))
                return accs

            accs = lax.fori_loop(
                0, H // UNROLL, body,
                tuple(jnp.zeros((L,), jnp.float32) for _ in range(D // L)))
            for j in range(D // L):
                out_v[e, pl.ds(j * L, L)] = accs[j]

        issue(0, rows0, sem0)

        @pl.loop(0, BPW // 2)
        def _(p):
            e0 = p * 2
            issue(e0 + 1, rows1, sem1)
            drain(e0, rows0, sem0)
            compute(e0, rows0)
            issue(jnp.minimum(e0 + 2, BPW - 1), rows0, sem0)
            drain(e0 + 1, rows1, sem1)
            compute(e0 + 1, rows1)

        drain(BPW - 1, rows0, sem0)

        pltpu.sync_copy(out_v, ctxsum_hbm.at[pl.ds(base, BPW)])

        pltpu.sync_copy(itm_hbm.at[pl.ds(base, BPW)], itmidx_v)
        pltpu.sync_copy(tab_hbm.at[itmidx_v], itmrows_v)
        pltpu.sync_copy(itmrows_v, itmrows_hbm.at[pl.ds(base, BPW)])

    return k(ctx, ctx_v, itm_flat, tab2)


def _tc_tail(ctx_sum, itm_rows, z, fc1_w, fc1_b):
    def body(cs_ref, it_ref, z_ref, w_ref, b_ref, o_ref):
        p = (cs_ref[...] * it_ref[:, :D] * w_ref[:, :D]
             + z_ref[...] * w_ref[:, D:])
        o_ref[...] = jnp.sum(p, axis=1, keepdims=True) + b_ref[...]

    return pl.pallas_call(
        body,
        out_shape=jax.ShapeDtypeStruct((B, 1), jnp.float32),
    )(ctx_sum, itm_rows, z, fc1_w, fc1_b)


def kernel(ctx, itm, pos, ctx_v, z, embed_w, fc1_w, fc1_b):
    del pos
    tab2 = _tc_widen_table(embed_w)
    ctx_sum, itm_rows = _sc_embedding_bag(ctx, ctx_v, itm.reshape(B), tab2)
    return _tc_tail(ctx_sum, itm_rows, z, fc1_w, fc1_b.reshape(1, 1))


# speedup vs baseline: 1.7573x; 1.1374x over previous
import dataclasses
import functools

import jax
import jax.numpy as jnp
from jax import lax
from jax.experimental import pallas as pl
from jax.experimental.pallas import tpu as pltpu
from jax.experimental.pallas import tpu_sc as plsc

NC = 2
NS = 16
L = 16
NW = NC * NS
B = 4096
H = 200
D = 64
W = 2 * D
BPW = B // NW
G0 = 128
G1 = H - G0
UNROLL = 8
REPACK_BLK = 8192


def _sc_compiler_params():
    cp = pltpu.CompilerParams()
    fields = pltpu.CompilerParams.__dataclass_fields__
    if "needs_layout_passes" in fields:
        cp = dataclasses.replace(cp, needs_layout_passes=False)
    if "use_tc_tiling_on_sc" in fields:
        cp = dataclasses.replace(cp, use_tc_tiling_on_sc=False)
    return cp


def _tc_widen_table(embed_w):
    v = embed_w.shape[0]
    tab_t = embed_w.T

    def body(in_ref, o_ref):
        o_ref[:, :D] = in_ref[...].T

    return pl.pallas_call(
        body,
        grid=((v + REPACK_BLK - 1) // REPACK_BLK,),
        in_specs=[pl.BlockSpec((D, REPACK_BLK), lambda i: (0, i))],
        out_specs=pl.BlockSpec((REPACK_BLK, W), lambda i: (i, 0)),
        out_shape=jax.ShapeDtypeStruct((v, W), jnp.float32),
    )(tab_t)


def _sc_embedding_bag(ctx, ctx_v, itm_flat, tab2):
    mesh = plsc.VectorSubcoreMesh(core_axis_name="c", subcore_axis_name="s")

    @functools.partial(
        pl.kernel,
        out_type=[jax.ShapeDtypeStruct((B, D), jnp.int32),
                  jax.ShapeDtypeStruct((B, W), jnp.float32)],
        mesh=mesh,
        compiler_params=_sc_compiler_params(),
        scratch_types=[
            pltpu.VMEM((BPW, H), jnp.int32),
            pltpu.VMEM((BPW, H), jnp.float32),
            pltpu.VMEM((H, W), jnp.float32),
            pltpu.VMEM((H, W), jnp.float32),
            pltpu.VMEM((H, W), jnp.float32),
            pltpu.VMEM((BPW,), jnp.int32),
            pltpu.SemaphoreType.DMA,
            pltpu.SemaphoreType.DMA,
            pltpu.SemaphoreType.DMA,
        ],
    )
    def k(ctx_hbm, ctxv_hbm, itm_hbm, tab_hbm, ctxsum_hbm, itmrows_hbm,
          idx_v, w_v, rows0, rows1, rows2, itmidx_v, sem0, sem1, sem2):
        wid = lax.axis_index("s") * NC + lax.axis_index("c")
        base = wid * BPW
        bufs = (rows0, rows1, rows2)
        sems = (sem0, sem1, sem2)

        pltpu.sync_copy(itm_hbm.at[pl.ds(base, BPW)], itmidx_v)

        pltpu.sync_copy(ctx_hbm.at[pl.ds(base, BPW)], idx_v)
        pltpu.sync_copy(ctxv_hbm.at[pl.ds(base, BPW)], w_v)

        def issue(e, buf, sem):
            pltpu.make_async_copy(
                tab_hbm.at[idx_v.at[e, pl.ds(0, G0)]],
                buf.at[pl.ds(0, G0)], sem).start()
            pltpu.make_async_copy(
                tab_hbm.at[idx_v.at[e, pl.ds(G0, G1)]],
                buf.at[pl.ds(G0, G1)], sem).start()

        def drain(e, buf, sem):
            pltpu.make_async_copy(
                tab_hbm.at[idx_v.at[e, pl.ds(0, G0)]],
                buf.at[pl.ds(0, G0)], sem).wait()
            pltpu.make_async_copy(
                tab_hbm.at[idx_v.at[e, pl.ds(G0, G1)]],
                buf.at[pl.ds(G0, G1)], sem).wait()

        def compute(e, buf):
            def body(l0, accs):
                for u in range(UNROLL):
                    l = l0 * UNROLL + u
                    wv = plsc.load_gather(
                        w_v, [jnp.full((L,), e, jnp.int32),
                              jnp.full((L,), l, jnp.int32)])
                    accs = tuple(acc + wv * buf[l, pl.ds(j * L, L)]
                                 for j, acc in enumerate(accs))
                return accs

            accs = lax.fori_loop(
                0, H // UNROLL, body,
                tuple(jnp.zeros((L,), jnp.float32) for _ in range(D // L)))
            for j in range(D // L):
                idx_v[e, pl.ds(j * L, L)] = plsc.bitcast(accs[j], jnp.int32)

        issue(0, bufs[0], sems[0])
        issue(1, bufs[1], sems[1])

        @pl.loop(0, BPW // 3)
        def _(p):
            e0 = p * 3
            for u in range(3):
                e = e0 + u
                drain(e, bufs[u], sems[u])
                compute(e, bufs[u])
                nxt = (u + 2) % 3
                issue(e + 2, bufs[nxt], sems[nxt])

        drain(BPW - 2, bufs[0], sems[0])
        compute(BPW - 2, bufs[0])
        drain(BPW - 1, bufs[1], sems[1])
        compute(BPW - 1, bufs[1])

        pltpu.sync_copy(idx_v.at[:, pl.ds(0, D)],
                        ctxsum_hbm.at[pl.ds(base, BPW)])

        pltpu.sync_copy(tab_hbm.at[itmidx_v], rows0.at[pl.ds(0, BPW)])
        pltpu.sync_copy(rows0.at[pl.ds(0, BPW)],
                        itmrows_hbm.at[pl.ds(base, BPW)])

    return k(ctx, ctx_v, itm_flat, tab2)


def _tc_tail(ctx_sum, itm_rows, z, fc1_w, fc1_b):
    def body(cs_ref, it_ref, z_ref, w_ref, b_ref, o_ref):
        cs = lax.bitcast_convert_type(cs_ref[...], jnp.float32)
        p = (cs * it_ref[:, :D] * w_ref[:, :D]
             + z_ref[...] * w_ref[:, D:])
        o_ref[...] = jnp.sum(p, axis=1, keepdims=True) + b_ref[...]

    return pl.pallas_call(
        body,
        out_shape=jax.ShapeDtypeStruct((B, 1), jnp.float32),
    )(ctx_sum, itm_rows, z, fc1_w, fc1_b)


def kernel(ctx, itm, pos, ctx_v, z, embed_w, fc1_w, fc1_b):
    del pos
    tab2 = _tc_widen_table(embed_w)
    ctx_sum, itm_rows = _sc_embedding_bag(ctx, ctx_v, itm.reshape(B), tab2)
    return _tc_tail(ctx_sum, itm_rows, z, fc1_w, fc1_b.reshape(1, 1))


# speedup vs baseline: 1.7649x; 1.0044x over previous
import dataclasses
import functools

import jax
import jax.numpy as jnp
from jax import lax
from jax.experimental import pallas as pl
from jax.experimental.pallas import tpu as pltpu
from jax.experimental.pallas import tpu_sc as plsc

NC = 2
NS = 16
L = 16
NW = NC * NS
B = 4096
H = 200
D = 64
W = 2 * D
BPW = B // NW
G0 = 128
G1 = H - G0
UNROLL = 8
REPACK_BLK = 8192


def _sc_compiler_params():
    cp = pltpu.CompilerParams()
    fields = pltpu.CompilerParams.__dataclass_fields__
    if "needs_layout_passes" in fields:
        cp = dataclasses.replace(cp, needs_layout_passes=False)
    if "use_tc_tiling_on_sc" in fields:
        cp = dataclasses.replace(cp, use_tc_tiling_on_sc=False)
    return cp


def _tc_widen_table(embed_w):
    v = embed_w.shape[0]
    tab_t = embed_w.T

    def body(in_ref, o_ref):
        o_ref[:, :D] = in_ref[...].T

    return pl.pallas_call(
        body,
        grid=((v + REPACK_BLK - 1) // REPACK_BLK,),
        in_specs=[pl.BlockSpec((D, REPACK_BLK), lambda i: (0, i))],
        out_specs=pl.BlockSpec((REPACK_BLK, W), lambda i: (i, 0)),
        out_shape=jax.ShapeDtypeStruct((v, W), jnp.float32),
    )(tab_t)


def _sc_embedding_bag(ctx, ctx_v, itm_flat, tab2):
    mesh = plsc.VectorSubcoreMesh(core_axis_name="c", subcore_axis_name="s")

    @functools.partial(
        pl.kernel,
        out_type=[jax.ShapeDtypeStruct((B, D), jnp.int32),
                  jax.ShapeDtypeStruct((B, W), jnp.float32)],
        mesh=mesh,
        compiler_params=_sc_compiler_params(),
        scratch_types=[
            pltpu.VMEM((BPW, H), jnp.int32),
            pltpu.VMEM((BPW, H), jnp.float32),
            pltpu.VMEM((H, W), jnp.float32),
            pltpu.VMEM((H, W), jnp.float32),
            pltpu.VMEM((H, W), jnp.float32),
            pltpu.VMEM((BPW,), jnp.int32),
            pltpu.SemaphoreType.DMA,
            pltpu.SemaphoreType.DMA,
            pltpu.SemaphoreType.DMA,
        ],
    )
    def k(ctx_hbm, ctxv_hbm, itm_hbm, tab_hbm, ctxsum_hbm, itmrows_hbm,
          idx_v, w_v, rows0, rows1, rows2, itmidx_v, sem0, sem1, sem2):
        wid = lax.axis_index("s") * NC + lax.axis_index("c")
        base = wid * BPW
        bufs = (rows0, rows1, rows2)
        sems = (sem0, sem1, sem2)

        pltpu.sync_copy(itm_hbm.at[pl.ds(base, BPW)], itmidx_v)

        pltpu.sync_copy(ctx_hbm.at[pl.ds(base, BPW)], idx_v)
        pltpu.sync_copy(ctxv_hbm.at[pl.ds(base, BPW)], w_v)

        def issue(e, buf, sem):
            pltpu.make_async_copy(
                tab_hbm.at[idx_v.at[e, pl.ds(0, G0)]],
                buf.at[pl.ds(0, G0)], sem).start()
            pltpu.make_async_copy(
                tab_hbm.at[idx_v.at[e, pl.ds(G0, G1)]],
                buf.at[pl.ds(G0, G1)], sem).start()

        def drain(e, buf, sem):
            pltpu.make_async_copy(
                tab_hbm.at[idx_v.at[e, pl.ds(0, G0)]],
                buf.at[pl.ds(0, G0)], sem).wait()
            pltpu.make_async_copy(
                tab_hbm.at[idx_v.at[e, pl.ds(G0, G1)]],
                buf.at[pl.ds(G0, G1)], sem).wait()

        def compute(e, buf):
            def fma16(accs, wvec, lbase, ubase):
                for u in range(L - ubase):
                    l = lbase + u
                    wu = lax.gather(
                        wvec, jnp.full((L, 1), ubase + u, jnp.int32),
                        dimension_numbers=lax.GatherDimensionNumbers(
                            offset_dims=(), collapsed_slice_dims=(0,),
                            start_index_map=(0,)),
                        slice_sizes=(1,),
                        mode=lax.GatherScatterMode.PROMISE_IN_BOUNDS)
                    accs = tuple(acc + wu * buf[l, pl.ds(j * L, L)]
                                 for j, acc in enumerate(accs))
                return accs

            def body(l0, accs):
                wvec = w_v[e, pl.ds(l0 * L, L)]
                return fma16(accs, wvec, l0 * L, 0)

            accs = lax.fori_loop(
                0, H // L, body,
                tuple(jnp.zeros((L,), jnp.float32) for _ in range(D // L)))
            wvec = w_v[e, pl.ds(H - L, L)]
            accs = fma16(accs, wvec, (H // L) * L, L - H % L)
            for j in range(D // L):
                idx_v[e, pl.ds(j * L, L)] = plsc.bitcast(accs[j], jnp.int32)

        issue(0, bufs[0], sems[0])
        issue(1, bufs[1], sems[1])

        @pl.loop(0, BPW // 3)
        def _(p):
            e0 = p * 3
            for u in range(3):
                e = e0 + u
                drain(e, bufs[u], sems[u])
                compute(e, bufs[u])
                nxt = (u + 2) % 3
                issue(e + 2, bufs[nxt], sems[nxt])

        drain(BPW - 2, bufs[0], sems[0])
        compute(BPW - 2, bufs[0])
        drain(BPW - 1, bufs[1], sems[1])
        compute(BPW - 1, bufs[1])

        pltpu.sync_copy(idx_v.at[:, pl.ds(0, D)],
                        ctxsum_hbm.at[pl.ds(base, BPW)])

        pltpu.sync_copy(tab_hbm.at[itmidx_v], rows0.at[pl.ds(0, BPW)])
        pltpu.sync_copy(rows0.at[pl.ds(0, BPW)],
                        itmrows_hbm.at[pl.ds(base, BPW)])

    return k(ctx, ctx_v, itm_flat, tab2)


def _tc_tail(ctx_sum, itm_rows, z, fc1_w, fc1_b):
    def body(cs_ref, it_ref, z_ref, w_ref, b_ref, o_ref):
        cs = lax.bitcast_convert_type(cs_ref[...], jnp.float32)
        p = (cs * it_ref[:, :D] * w_ref[:, :D]
             + z_ref[...] * w_ref[:, D:])
        o_ref[...] = jnp.sum(p, axis=1, keepdims=True) + b_ref[...]

    return pl.pallas_call(
        body,
        out_shape=jax.ShapeDtypeStruct((B, 1), jnp.float32),
    )(ctx_sum, itm_rows, z, fc1_w, fc1_b)


def kernel(ctx, itm, pos, ctx_v, z, embed_w, fc1_w, fc1_b):
    del pos
    tab2 = _tc_widen_table(embed_w)
    ctx_sum, itm_rows = _sc_embedding_bag(ctx, ctx_v, itm.reshape(B), tab2)
    return _tc_tail(ctx_sum, itm_rows, z, fc1_w, fc1_b.reshape(1, 1))
